# Initial kernel scaffold; baseline (speedup 1.0000x reference)
import jax
import jax.numpy as jnp
from jax.experimental import pallas as pl


def kernel(image, grid, center, size, xlors, ylors, zlors):
    raise NotImplementedError("write your pallas kernel here")


# speedup vs baseline: 252.3959x; 252.3959x over previous
import functools

import jax
import jax.numpy as jnp
import numpy as np
from jax import lax
from jax.experimental import pallas as pl
from jax.experimental.pallas import tpu as pltpu
from jax.experimental.pallas import tpu_sc as plsc

KW = float(np.sqrt(3.0 * 3.0 / np.pi))
NS = 64
C = 784
GROUPS = C // 16
INV63 = np.float32(1.0 / 63.0)


def _sc_projection(Q, params, scales, img_flat, strides, dims, n_chunks, Mp):
    T = Mp // 32
    info = plsc.get_sparse_core_info()
    NC = info.num_cores

    mesh = plsc.VectorSubcoreMesh(core_axis_name="c", subcore_axis_name="s")

    @functools.partial(
        pl.kernel,
        mesh=mesh,
        out_type=jax.ShapeDtypeStruct((3 * Mp,), jnp.float32),
        scratch_types=[
            pltpu.VMEM((6 * C,), jnp.float32),
            pltpu.VMEM((96,), jnp.float32),
            pltpu.VMEM((C * NS,), jnp.int32),
            pltpu.VMEM((C * NS,), jnp.float32),
            pltpu.VMEM((C,), jnp.float32),
            pltpu.VMEM((C,), jnp.float32),
            pltpu.SemaphoreType.DMA,
        ],
    )
    def proj_kernel(q_hbm, par_hbm, scl_hbm, img_hbm, out_hbm,
                    qv, pv, idxv, valv, sclv, outv, sem):
        wid = lax.axis_index("s") * NC + lax.axis_index("c")

        for s in range(3):
            sa, sb, sc_ = strides[s]
            da_max, db_max, dc_max = dims[s][0] - 1, dims[s][1] - 1, dims[s][2] - 1
            pltpu.sync_copy(par_hbm.at[pl.ds(s * 96, 96)], pv)
            org_a = pv[pl.ds(0, 16)]
            org_b = pv[pl.ds(16, 16)]
            org_c = pv[pl.ds(32, 16)]
            vox_a = pv[pl.ds(48, 16)]
            vox_b = pv[pl.ds(64, 16)]
            vox_c = pv[pl.ds(80, 16)]

            def chunk_body(ch, _, s=s, sa=sa, sb=sb, sc_=sc_,
                           da_max=da_max, db_max=db_max, dc_max=dc_max,
                           org_a=org_a, org_b=org_b, org_c=org_c,
                           vox_a=vox_a, vox_b=vox_b, vox_c=vox_c):
                off = wid * T + ch * C
                for comp in range(6):
                    pltpu.sync_copy(
                        q_hbm.at[pl.ds((s * 6 + comp) * Mp + off, C)],
                        qv.at[pl.ds(comp * C, C)])
                pltpu.sync_copy(scl_hbm.at[pl.ds(s * Mp + off, C)], sclv)

                def idx_body(g, _):
                    p0a = qv[pl.ds(0 * C + g * 16, 16)]
                    p0b = qv[pl.ds(1 * C + g * 16, 16)]
                    p0c = qv[pl.ds(2 * C + g * 16, 16)]
                    dda = qv[pl.ds(3 * C + g * 16, 16)]
                    ddb = qv[pl.ds(4 * C + g * 16, 16)]
                    ddc = qv[pl.ds(5 * C + g * 16, 16)]
                    base = g * (16 * NS)

                    def kk_body(kk, _):
                        for k8 in range(8):
                            k = kk * 8 + k8
                            t = k.astype(jnp.float32) * INV63
                            ia = jnp.clip(
                                ((p0a + dda * t - org_a) / vox_a
                                 ).astype(jnp.int32), 0, da_max)
                            ib = jnp.clip(
                                ((p0b + ddb * t - org_b) / vox_b
                                 ).astype(jnp.int32), 0, db_max)
                            ic = jnp.clip(
                                ((p0c + ddc * t - org_c) / vox_c
                                 ).astype(jnp.int32), 0, dc_max)
                            flat = ia * sa + ib * sb + ic * sc_
                            idxv[pl.ds(base + k * 16, 16)] = flat
                        return 0

                    lax.fori_loop(0, 8, kk_body, 0)
                    return 0

                lax.fori_loop(0, GROUPS, idx_body, 0)

                pltpu.async_copy(img_hbm.at[idxv], valv, sem).wait()

                def red_body(g, _):
                    base = g * (16 * NS)
                    acc0 = valv[pl.ds(base + 0 * 16, 16)]
                    acc1 = valv[pl.ds(base + 1 * 16, 16)]
                    acc2 = valv[pl.ds(base + 2 * 16, 16)]
                    acc3 = valv[pl.ds(base + 3 * 16, 16)]
                    for k in range(4, NS, 4):
                        acc0 = acc0 + valv[pl.ds(base + k * 16, 16)]
                        acc1 = acc1 + valv[pl.ds(base + (k + 1) * 16, 16)]
                        acc2 = acc2 + valv[pl.ds(base + (k + 2) * 16, 16)]
                        acc3 = acc3 + valv[pl.ds(base + (k + 3) * 16, 16)]
                    total = (acc0 + acc1) + (acc2 + acc3)
                    outv[pl.ds(g * 16, 16)] = total * sclv[pl.ds(g * 16, 16)]
                    return 0

                lax.fori_loop(0, GROUPS, red_body, 0)
                pltpu.sync_copy(outv, out_hbm.at[pl.ds(s * Mp + off, C)])
                return 0

            lax.fori_loop(0, n_chunks, chunk_body, 0)

    return proj_kernel(Q, params, scales, img_flat)


def kernel(image, grid, center, size, xlors, ylors, zlors):
    X, Y, Z = image.shape
    sx, sy, sz = Y * Z, Z, 1
    perms = ((2, 0, 1), (1, 0, 2), (0, 1, 2))
    strides = ((sz, sx, sy), (sy, sx, sz), (sx, sy, sz))
    dims_all = (X, Y, Z)
    dims = tuple(tuple(dims_all[p] for p in perm) for perm in perms)

    lors_sets = (xlors, ylors, zlors)
    M = max(l.shape[0] for l in lors_sets)
    n_chunks = -(-M // (32 * C))
    Mp = 32 * C * n_chunks

    voxel = size / grid
    origin = center - size / 2.0

    Qs, scls, pars = [], [], []
    for lors, perm in zip(lors_sets, perms):
        p0 = lors[:, 0:3]
        d = lors[:, 3:6] - p0
        scale = jnp.sqrt(jnp.sum(d * d, axis=-1)) * (KW / NS)
        comp = jnp.concatenate([p0, d], axis=1).T
        pad = Mp - lors.shape[0]
        Qs.append(jnp.pad(comp, ((0, 0), (0, pad))))
        scls.append(jnp.pad(scale, (0, pad)))
        o = jnp.stack([origin[perm[0]], origin[perm[1]], origin[perm[2]]])
        v = jnp.stack([voxel[perm[0]], voxel[perm[1]], voxel[perm[2]]])
        pars.append(jnp.broadcast_to(
            jnp.concatenate([o, v])[:, None], (6, 16)))

    Q = jnp.stack(Qs).reshape(-1)
    scales = jnp.stack(scls).reshape(-1)
    params = jnp.stack(pars).reshape(-1)
    img_flat = image.reshape(-1)

    out = _sc_projection(Q, params, scales, img_flat, strides, dims,
                         n_chunks, Mp)
    return (out[0 * Mp:0 * Mp + xlors.shape[0]],
            out[1 * Mp:1 * Mp + ylors.shape[0]],
            out[2 * Mp:2 * Mp + zlors.shape[0]])
